# baseline (speedup 1.0000x reference)
import jax
import jax.numpy as jnp
from jax import lax
from jax.experimental import pallas as pl
from jax.experimental.pallas import tpu as pltpu
from jax.experimental.pallas import tpu_sc as plsc

N_NODES = 10000
N_PAD = 10240
N_EDGES = 320000
IN_CH = 128
HID_CH = 128
OUT_CH = 64

NC = 2
NS = 16
NW = NC * NS
CHUNK = 128
TOT_CH = 2560
E_PAD = TOT_CH * CHUNK
NCH_AGG = TOT_CH // NW
NCH_DEG = TOT_CH // NW
ROWS_PER_TILE = N_PAD // NS
IBLK = 40

_MESH = plsc.VectorSubcoreMesh(core_axis_name="c", subcore_axis_name="s")


def _deg_body(dst_hbm, ones_hbm, zeros_hbm, out_hbm, idx_v, ones_v, zrow_v,
              deg_sh):
    c = lax.axis_index("c")
    s = lax.axis_index("s")
    wid = c * NS + s
    pltpu.sync_copy(ones_hbm, ones_v)
    pltpu.sync_copy(zeros_hbm, zrow_v)
    pltpu.sync_copy(zrow_v, deg_sh.at[pl.ds(s * ROWS_PER_TILE, ROWS_PER_TILE)])
    pltpu.sync_copy(dst_hbm.at[pl.ds(wid * NCH_DEG, NCH_DEG)], idx_v)
    plsc.subcore_barrier()

    def body(i, carry):
        pltpu.sync_copy(ones_v, deg_sh.at[idx_v.at[i]], add=True)
        return carry

    lax.fori_loop(0, NCH_DEG, body, 0)
    plsc.subcore_barrier()
    pltpu.sync_copy(deg_sh.at[pl.ds(s * ROWS_PER_TILE, ROWS_PER_TILE)], zrow_v)
    pltpu.sync_copy(zrow_v, out_hbm.at[c, pl.ds(s * ROWS_PER_TILE, ROWS_PER_TILE)])


def _deg_partials(dst_r, ones_c, zeros_r):
    return pl.kernel(
        _deg_body,
        out_type=jax.ShapeDtypeStruct((NC, N_PAD), jnp.float32),
        mesh=_MESH,
        scratch_types=[
            pltpu.VMEM((NCH_DEG, CHUNK), jnp.int32),
            pltpu.VMEM((CHUNK,), jnp.float32),
            pltpu.VMEM((ROWS_PER_TILE,), jnp.float32),
            pltpu.VMEM_SHARED((N_PAD,), jnp.float32),
        ],
    )(dst_r, ones_c, zeros_r)


def _agg_body(y_hbm, src_hbm, dst_hbm, zeros_hbm, out_hbm, srcv, dstv, rows,
              rows2, sem_a, sem_b, acc_sh):
    c = lax.axis_index("c")
    s = lax.axis_index("s")
    base = (c * NS + s) * NCH_AGG
    pltpu.sync_copy(zeros_hbm, rows)
    for j in range(5):
        pltpu.sync_copy(rows, acc_sh.at[pl.ds(s * ROWS_PER_TILE + j * 128, 128)])
    plsc.subcore_barrier()

    def blk_body(blk, carry):
        b0 = base + blk * IBLK
        pltpu.sync_copy(src_hbm.at[pl.ds(b0, IBLK)], srcv)
        pltpu.sync_copy(dst_hbm.at[pl.ds(b0, IBLK)], dstv)

        pltpu.async_copy(y_hbm.at[srcv.at[0]], rows, sem_a)

        def pair(ii, carry2):
            i0 = 2 * ii
            i_next = jnp.minimum(i0 + 2, IBLK - 1)
            pltpu.async_copy(y_hbm.at[srcv.at[i0 + 1]], rows2, sem_b)
            pltpu.make_async_copy(y_hbm.at[srcv.at[0]], rows, sem_a).wait()
            pltpu.sync_copy(rows, acc_sh.at[dstv.at[i0]], add=True)
            pltpu.async_copy(y_hbm.at[srcv.at[i_next]], rows, sem_a)
            pltpu.make_async_copy(y_hbm.at[srcv.at[0]], rows2, sem_b).wait()
            pltpu.sync_copy(rows2, acc_sh.at[dstv.at[i0 + 1]], add=True)
            return carry2

        lax.fori_loop(0, IBLK // 2, pair, 0)
        pltpu.make_async_copy(y_hbm.at[srcv.at[0]], rows, sem_a).wait()
        return carry

    lax.fori_loop(0, NCH_AGG // IBLK, blk_body, 0)
    plsc.subcore_barrier()
    for j in range(5):
        r0 = s * ROWS_PER_TILE + j * 128
        pltpu.sync_copy(acc_sh.at[pl.ds(r0, 128)], rows)
        pltpu.sync_copy(rows, out_hbm.at[c, pl.ds(r0, 128)])


def _agg_partials(y, src_r, dst_r, zeros_b):
    return pl.kernel(
        _agg_body,
        out_type=jax.ShapeDtypeStruct((NC, N_PAD, HID_CH), jnp.float32),
        mesh=_MESH,
        scratch_types=[
            pltpu.VMEM((IBLK, CHUNK), jnp.int32),
            pltpu.VMEM((IBLK, CHUNK), jnp.int32),
            pltpu.VMEM((CHUNK, HID_CH), jnp.float32),
            pltpu.VMEM((CHUNK, HID_CH), jnp.float32),
            pltpu.SemaphoreType.DMA,
            pltpu.SemaphoreType.DMA,
            pltpu.VMEM_SHARED((N_PAD, HID_CH), jnp.float32),
        ],
    )(y, src_r, dst_r, zeros_b)


def _dis(dp_ref):
    deg = dp_ref[:, 0:1] + dp_ref[:, 1:2] + 1.0
    return lax.rsqrt(deg)


def _tc1_body(x_ref, w_ref, dp_ref, y_ref):
    dis = _dis(dp_ref)
    y_ref[:N_NODES, :] = jnp.dot(x_ref[...], w_ref[...],
                                 preferred_element_type=jnp.float32) * dis


def _tc2_body(p_ref, y0_ref, dp_ref, b_ref, w_ref, y1_ref):
    dis = _dis(dp_ref)
    agg = (p_ref[0, :N_NODES, :] + p_ref[1, :N_NODES, :]
           + y0_ref[:N_NODES, :])
    h = jnp.maximum(agg * dis + b_ref[...], 0.0)
    y1_ref[:N_NODES, :] = jnp.dot(h, w_ref[...],
                                  preferred_element_type=jnp.float32) * dis


def _tc3_body(p_ref, y1_ref, dp_ref, b_ref, wm_ref, bm_ref, wl_ref, bl_ref,
              mean_ref, logstd_ref):
    dis = _dis(dp_ref)
    agg = (p_ref[0, :N_NODES, :] + p_ref[1, :N_NODES, :]
           + y1_ref[:N_NODES, :])
    h = jnp.maximum(agg * dis + b_ref[...], 0.0)
    mean_ref[...] = jnp.dot(h, wm_ref[...],
                            preferred_element_type=jnp.float32) + bm_ref[...]
    logstd_ref[...] = jnp.dot(h, wl_ref[...],
                              preferred_element_type=jnp.float32) + bl_ref[...]


_ytbl = jax.ShapeDtypeStruct((N_PAD, HID_CH), jnp.float32)
_tc1 = pl.pallas_call(_tc1_body, out_shape=_ytbl)
_tc2 = pl.pallas_call(_tc2_body, out_shape=_ytbl)
_tc3 = pl.pallas_call(
    _tc3_body, out_shape=(jax.ShapeDtypeStruct((N_NODES, OUT_CH), jnp.float32),
                          jax.ShapeDtypeStruct((N_NODES, OUT_CH), jnp.float32)))


def kernel(x, edge_index, W0, b0, W1, b1, Wm, bm, Wl, bl):
    src = edge_index[0].astype(jnp.int32)
    dst = edge_index[1].astype(jnp.int32)
    npad = E_PAD - N_EDGES
    src_r = jnp.concatenate([src, jnp.zeros((npad,), jnp.int32)])
    src_r = src_r.reshape(TOT_CH, CHUNK)
    junk = N_NODES + (jnp.arange(npad, dtype=jnp.int32) % (N_PAD - N_NODES))
    dst_r = jnp.concatenate([dst, junk]).reshape(TOT_CH, CHUNK)
    ones_c = jnp.ones((CHUNK,), jnp.float32)
    zeros_r = jnp.zeros((ROWS_PER_TILE,), jnp.float32)
    zeros_b = jnp.zeros((CHUNK, HID_CH), jnp.float32)

    dp = _deg_partials(dst_r, ones_c, zeros_r)
    dpt = dp[:, :N_NODES].T

    y0 = _tc1(x, W0, dpt)
    p0 = _agg_partials(y0, src_r, dst_r, zeros_b)
    y1 = _tc2(p0, y0, dpt, b0, W1)
    p1 = _agg_partials(y1, src_r, dst_r, zeros_b)
    mean, logstd = _tc3(p1, y1, dpt, b1, Wm, bm, Wl, bl)
    return (mean, logstd)
